# baseline (speedup 1.0000x reference)
import functools
import jax
import jax.numpy as jnp
from jax.experimental import pallas as pl
from jax.experimental.pallas import tpu as pltpu

_NEG = -1e30


def _qkv_body(x_ref, nw_ref, qw_ref, kw_ref, vw_ref, q_ref, k_ref, v_ref):
    xb = x_ref[...]
    var = jnp.mean(xb * xb, axis=1, keepdims=True)
    h = (nw_ref[...] * xb) * jax.lax.rsqrt(var + 1e-6)
    q_ref[...] = jnp.dot(h, qw_ref[...], preferred_element_type=jnp.float32)
    k_ref[...] = jnp.dot(h, kw_ref[...], preferred_element_type=jnp.float32)
    v_ref[...] = jnp.dot(h, vw_ref[...], preferred_element_type=jnp.float32)


def _qkv(xf, attn_norm_w, q_w, k_w, v_w, tm):
    n, c = xf.shape
    grid = (n // tm,)
    wspec = pl.BlockSpec((c, c), lambda i: (0, 0))
    return pl.pallas_call(
        _qkv_body,
        grid=grid,
        in_specs=[
            pl.BlockSpec((tm, c), lambda i: (i, 0)),
            pl.BlockSpec((1, c), lambda i: (0, 0)),
            wspec, wspec, wspec,
        ],
        out_specs=[pl.BlockSpec((tm, c), lambda i: (i, 0))] * 3,
        out_shape=[jax.ShapeDtypeStruct((n, c), jnp.float32)] * 3,
        compiler_params=pltpu.CompilerParams(
            dimension_semantics=("parallel",)),
    )(xf, attn_norm_w.reshape(1, c), q_w, k_w, v_w)


def _rope(xb, cos, sin):
    hd = xb.shape[-1]
    x1 = xb[:, : hd // 2]
    x2 = xb[:, hd // 2:]
    rot = jnp.concatenate([-x2, x1], axis=-1)
    return xb * cos + rot * sin


def _flash_body(q_ref, k_ref, v_ref, cq_ref, sq_ref, ck_ref, sk_ref,
                o_ref, acc_ref, m_ref, l_ref, *, scale, tq, tk):
    i = pl.program_id(1)
    j = pl.program_id(2)

    @pl.when(j == 0)
    def _init():
        m_ref[...] = jnp.full_like(m_ref, _NEG)
        l_ref[...] = jnp.zeros_like(l_ref)
        acc_ref[...] = jnp.zeros_like(acc_ref)

    @pl.when(j <= i)
    def _compute():
        qb = _rope(q_ref[0], cq_ref[...], sq_ref[...])
        kb = _rope(k_ref[0], ck_ref[...], sk_ref[...])
        s = jax.lax.dot_general(qb, kb, (((1,), (1,)), ((), ())),
                                preferred_element_type=jnp.float32) * scale
        rows = jax.lax.broadcasted_iota(jnp.int32, (tq, tk), 0)
        cols = jax.lax.broadcasted_iota(jnp.int32, (tq, tk), 1)
        s = jnp.where((j < i) | (rows >= cols), s, _NEG)

        m_prev = m_ref[...][:, :1]
        l_prev = l_ref[...][:, :1]
        m_new = jnp.maximum(m_prev, jnp.max(s, axis=1, keepdims=True))
        alpha = jnp.exp(m_prev - m_new)
        p = jnp.exp(s - m_new)
        l_new = alpha * l_prev + jnp.sum(p, axis=1, keepdims=True)
        acc_ref[...] = alpha * acc_ref[...] + jnp.dot(
            p, v_ref[0], preferred_element_type=jnp.float32)
        m_ref[...] = jnp.broadcast_to(m_new, m_ref.shape)
        l_ref[...] = jnp.broadcast_to(l_new, l_ref.shape)

    @pl.when(j == i)
    def _finish():
        o_ref[0] = acc_ref[...] / l_ref[...][:, :1]


def _flash(q, k, v, cos, sin, tq, tk):
    bh, t, dh = q.shape
    scale = 1.0 / (dh ** 0.5)
    grid = (bh, t // tq, t // tk)
    body = functools.partial(_flash_body, scale=scale, tq=tq, tk=tk)
    return pl.pallas_call(
        body,
        grid=grid,
        in_specs=[
            pl.BlockSpec((1, tq, dh), lambda b, i, j: (b, i, 0)),
            pl.BlockSpec((1, tk, dh), lambda b, i, j: (b, j, 0)),
            pl.BlockSpec((1, tk, dh), lambda b, i, j: (b, j, 0)),
            pl.BlockSpec((tq, dh), lambda b, i, j: (i, 0)),
            pl.BlockSpec((tq, dh), lambda b, i, j: (i, 0)),
            pl.BlockSpec((tk, dh), lambda b, i, j: (j, 0)),
            pl.BlockSpec((tk, dh), lambda b, i, j: (j, 0)),
        ],
        out_specs=pl.BlockSpec((1, tq, dh), lambda b, i, j: (b, i, 0)),
        out_shape=jax.ShapeDtypeStruct((bh, t, dh), jnp.float32),
        scratch_shapes=[
            pltpu.VMEM((tq, dh), jnp.float32),
            pltpu.VMEM((tq, 128), jnp.float32),
            pltpu.VMEM((tq, 128), jnp.float32),
        ],
        compiler_params=pltpu.CompilerParams(
            dimension_semantics=("parallel", "parallel", "arbitrary")),
    )(q, k, v, cos, sin, cos, sin)


def _post_body(x_ref, y_ref, ow_ref, fw_ref, rw_ref,
               x2_ref, hm_ref, comb_ref, idx_ref, wt_ref, *, e):
    xb = x_ref[...]
    x2 = xb + jnp.dot(y_ref[...], ow_ref[...],
                      preferred_element_type=jnp.float32)
    var = jnp.mean(x2 * x2, axis=1, keepdims=True)
    hm = (fw_ref[...] * x2) * jax.lax.rsqrt(var + 1e-6)
    logits = jnp.dot(hm, rw_ref[...], preferred_element_type=jnp.float32)
    tm, lanes = logits.shape
    colid = jax.lax.broadcasted_iota(jnp.int32, (tm, lanes), 1)
    lg = jnp.where(colid < e, logits, _NEG)
    m1 = jnp.max(lg, axis=1, keepdims=True)
    i1 = jnp.min(jnp.where(lg == m1, colid, 999), axis=1, keepdims=True)
    lg2 = jnp.where(colid == i1, _NEG, lg)
    m2 = jnp.max(lg2, axis=1, keepdims=True)
    i2 = jnp.min(jnp.where(lg2 == m2, colid, 999), axis=1, keepdims=True)
    e2 = jnp.exp(m2 - m1)
    w1v = 1.0 / (1.0 + e2)
    w2v = e2 / (1.0 + e2)
    x2_ref[...] = x2
    hm_ref[...] = hm
    comb_ref[...] = jnp.where(colid == i1, w1v,
                              jnp.where(colid == i2, w2v, 0.0))
    idx_ref[...] = jnp.where(colid == 0, i1, jnp.where(colid == 1, i2, 0))
    wt_ref[...] = jnp.where(colid == 0, w1v, jnp.where(colid == 1, w2v, 0.0))


def _post(xf, yf, o_w, ffn_norm_w, router_w, tm):
    n, c = xf.shape
    e = router_w.shape[1]
    rw = jnp.pad(router_w, ((0, 0), (0, 128 - e)))
    grid = (n // tm,)
    body = functools.partial(_post_body, e=e)
    return pl.pallas_call(
        body,
        grid=grid,
        in_specs=[
            pl.BlockSpec((tm, c), lambda i: (i, 0)),
            pl.BlockSpec((tm, c), lambda i: (i, 0)),
            pl.BlockSpec((c, c), lambda i: (0, 0)),
            pl.BlockSpec((1, c), lambda i: (0, 0)),
            pl.BlockSpec((c, 128), lambda i: (0, 0)),
        ],
        out_specs=[
            pl.BlockSpec((tm, c), lambda i: (i, 0)),
            pl.BlockSpec((tm, c), lambda i: (i, 0)),
            pl.BlockSpec((tm, 128), lambda i: (i, 0)),
            pl.BlockSpec((tm, 128), lambda i: (i, 0)),
            pl.BlockSpec((tm, 128), lambda i: (i, 0)),
        ],
        out_shape=[
            jax.ShapeDtypeStruct((n, c), jnp.float32),
            jax.ShapeDtypeStruct((n, c), jnp.float32),
            jax.ShapeDtypeStruct((n, 128), jnp.float32),
            jax.ShapeDtypeStruct((n, 128), jnp.int32),
            jax.ShapeDtypeStruct((n, 128), jnp.float32),
        ],
        compiler_params=pltpu.CompilerParams(
            dimension_semantics=("parallel",)),
    )(xf, yf, o_w, ffn_norm_w.reshape(1, c), rw)


def _moe_body(hm_ref, x2_ref, comb_ref, w1_ref, w3_ref, w2_ref, out_ref):
    e = pl.program_id(1)
    hmb = hm_ref[...]
    a = jnp.dot(hmb, w1_ref[0], preferred_element_type=jnp.float32)
    b3 = jnp.dot(hmb, w3_ref[0], preferred_element_type=jnp.float32)
    act = (a * (1.0 / (1.0 + jnp.exp(-a)))) * b3
    o = jnp.dot(act, w2_ref[0], preferred_element_type=jnp.float32)
    tm, lanes = comb_ref.shape
    colid = jax.lax.broadcasted_iota(jnp.int32, (tm, lanes), 1)
    wcol = jnp.sum(jnp.where(colid == e, comb_ref[...], 0.0),
                   axis=1, keepdims=True)

    @pl.when(e == 0)
    def _init():
        out_ref[...] = x2_ref[...] + wcol * o

    @pl.when(e > 0)
    def _acc():
        out_ref[...] += wcol * o


def _moe_dense(hm, x2, comb, w1, w2, w3, tm):
    n, c = hm.shape
    ne, _, hid = w1.shape
    grid = (n // tm, ne)
    return pl.pallas_call(
        _moe_body,
        grid=grid,
        in_specs=[
            pl.BlockSpec((tm, c), lambda m, e: (m, 0)),
            pl.BlockSpec((tm, c), lambda m, e: (m, 0)),
            pl.BlockSpec((tm, 128), lambda m, e: (m, 0)),
            pl.BlockSpec((1, c, hid), lambda m, e: (e, 0, 0)),
            pl.BlockSpec((1, c, hid), lambda m, e: (e, 0, 0)),
            pl.BlockSpec((1, hid, c), lambda m, e: (e, 0, 0)),
        ],
        out_specs=pl.BlockSpec((tm, c), lambda m, e: (m, 0)),
        out_shape=jax.ShapeDtypeStruct((n, c), jnp.float32),
        compiler_params=pltpu.CompilerParams(
            dimension_semantics=("parallel", "arbitrary")),
    )(hm, x2, comb, w1, w3, w2)


def kernel(x, rope_cos, rope_sin, attn_norm_w, q_w, k_w, v_w, o_w,
           ffn_norm_w, router_w, w1, w2, w3):
    b, t, c = x.shape
    dh = rope_cos.shape[1]
    h = c // dh
    n = b * t

    xf = x.reshape(n, c)
    tm = min(512, n)
    q, k, v = _qkv(xf, attn_norm_w, q_w, k_w, v_w, tm)

    def to_heads(a):
        return (a.reshape(b, t, h, dh).transpose(0, 2, 1, 3)
                .reshape(b * h, t, dh))

    tq = min(256, t)
    y = _flash(to_heads(q), to_heads(k), to_heads(v), rope_cos, rope_sin,
               tq, tq)
    yf = (y.reshape(b, h, t, dh).transpose(0, 2, 1, 3).reshape(n, c))

    x2, hm, comb, idx2, wt2 = _post(xf, yf, o_w, ffn_norm_w, router_w, tm)

    out = _moe_dense(hm, x2, comb, w1, w2, w3, tm)
    return out.reshape(b, t, c)
